# baseline (speedup 1.0000x reference)
import functools

import jax
import jax.numpy as jnp
from jax import lax
from jax.experimental import pallas as pl
from jax.experimental.pallas import tpu as pltpu
from jax.experimental.pallas import tpu_sc as plsc

B = 4
N = 10000
E = 320000
F_IN = 128
HID = 32
P = 12
OUT_F = 2

NT = 32
CPT = 320
CHUNK = 2048
GCH = 64
ROW_W = B * 2 * HID
E_PAD = E + NT * CHUNK

_mesh = plsc.VectorSubcoreMesh(core_axis_name="c", subcore_axis_name="s")
_sc_params = pltpu.CompilerParams(needs_layout_passes=False)


def _wid():
    return lax.axis_index("s") * 2 + lax.axis_index("c")


def _ds(start, size, mult=8):
    return pl.ds(pl.multiple_of(start, mult), size)


@functools.partial(
    pl.kernel,
    out_type=jax.ShapeDtypeStruct((N,), jnp.float32),
    mesh=_mesh,
    compiler_params=_sc_params,
    scratch_types=[
        pltpu.VMEM((40,), jnp.int32),
        pltpu.VMEM((CPT,), jnp.float32),
        pltpu.VMEM((CHUNK,), jnp.int32),
        pltpu.VMEM((CHUNK,), jnp.float32),
    ],
)
def _deg_kernel(cols_hbm, ew_hbm, tb_hbm, deg_hbm, tb_v, deg_v, cbuf,
                wbuf):
    wid = _wid()
    col0 = wid * CPT
    pltpu.sync_copy(tb_hbm, tb_v)
    lane = lax.iota(jnp.int32, 16)
    lane0 = lane == 0
    widv = jnp.broadcast_to(wid, (16,))
    tsv = plsc.load_gather(tb_v, [widv])
    tev = plsc.load_gather(tb_v, [widv + 1])
    ts = jnp.max(tsv)
    te = jnp.max(tev)

    for i in range(CPT // 16):
        deg_v[pl.ds(i * 16, 16)] = jnp.ones((16,), jnp.float32)

    def chunk_body(ci, _c):
        base = ci * CHUNK

        @pl.when((base + CHUNK > ts) & (base < te))
        def _():
            pltpu.sync_copy(cols_hbm.at[_ds(base, CHUNK, CHUNK)], cbuf)
            pltpu.sync_copy(ew_hbm.at[_ds(base, CHUNK, CHUNK)], wbuf)

            def grp(gi, _):
                g0 = gi * 16
                for k in range(16):
                    posv = jnp.broadcast_to(g0 + k, (16,))
                    ck = plsc.load_gather(cbuf, [posv]) - col0
                    wk = plsc.load_gather(wbuf, [posv])
                    gv = posv + base
                    mk = (gv >= tsv) & (gv < tev) & lane0
                    plsc.addupdate_scatter(deg_v, [ck], wk, mask=mk)
                return 0

            lax.fori_loop(0, CHUNK // 16, grp, 0)

        return 0

    lax.fori_loop(0, E_PAD // CHUNK, chunk_body, 0)

    @pl.when(wid < NT - 1)
    def _():
        pltpu.sync_copy(deg_v, deg_hbm.at[_ds(col0, CPT, CPT)])

    @pl.when(wid == NT - 1)
    def _():
        pltpu.sync_copy(deg_v.at[pl.ds(0, N - (NT - 1) * CPT)],
                        deg_hbm.at[_ds(col0, N - (NT - 1) * CPT, 16)])


@functools.partial(
    pl.kernel,
    out_type=jax.ShapeDtypeStruct((P * N, ROW_W), jnp.float32),
    mesh=_mesh,
    compiler_params=_sc_params,
    scratch_types=[
        pltpu.VMEM((40,), jnp.int32),
        pltpu.VMEM((CPT, ROW_W), jnp.float32),
        pltpu.VMEM((CHUNK,), jnp.int32),
        pltpu.VMEM((CHUNK,), jnp.int32),
        pltpu.VMEM((CHUNK,), jnp.float32),
        pltpu.VMEM((2, GCH), jnp.int32),
        pltpu.VMEM((2, GCH, ROW_W), jnp.float32),
        pltpu.SemaphoreType.DMA,
        pltpu.SemaphoreType.DMA,
    ],
)
def _agg_kernel(xw_hbm, rows_hbm, cols_hbm, ew_hbm, tb_hbm, agg_hbm,
                tb_v, acc_v, rbuf, cbuf, wbuf, idx2, gat2, sem0, sem1):
    wid = _wid()
    col0 = wid * CPT
    pltpu.sync_copy(tb_hbm, tb_v)
    lane = lax.iota(jnp.int32, 16)
    powv = jnp.int32(1) << lane
    zeros16 = jnp.zeros((16,), jnp.float32)
    widv = jnp.broadcast_to(wid, (16,))
    tsv = plsc.load_gather(tb_v, [widv])
    tev = plsc.load_gather(tb_v, [widv + 1])
    ts = jnp.max(tsv)
    te = jnp.max(tev)

    def per_period(p, _):
        pN = p * N

        def zrow(i, _z):
            for j in range(ROW_W // 16):
                acc_v[i, pl.ds(j * 16, 16)] = zeros16
            return 0

        lax.fori_loop(0, CPT, zrow, 0)

        def chunk_body(ci, _c):
            base = ci * CHUNK

            @pl.when((base + CHUNK > ts) & (base < te))
            def _():
                _chunk_inner(base)
            return 0

        def _fire(buf, s0):
            for q in range(GCH // 16):
                idx2[buf, pl.ds(q * 16, 16)] = (
                    rbuf[_ds(s0 + q * 16, 16, 16)] + pN)

            @pl.when(buf == 0)
            def _():
                pltpu.async_copy(xw_hbm.at[idx2.at[0]], gat2.at[0], sem0)

            @pl.when(buf == 1)
            def _():
                pltpu.async_copy(xw_hbm.at[idx2.at[1]], gat2.at[1], sem1)

        def _wait(buf):
            @pl.when(buf == 0)
            def _():
                pltpu.make_async_copy(
                    xw_hbm.at[idx2.at[0]], gat2.at[0], sem0).wait()

            @pl.when(buf == 1)
            def _():
                pltpu.make_async_copy(
                    xw_hbm.at[idx2.at[1]], gat2.at[1], sem1).wait()

        def _chunk_inner(base):
            pltpu.sync_copy(rows_hbm.at[_ds(base, CHUNK, CHUNK)], rbuf)
            pltpu.sync_copy(cols_hbm.at[_ds(base, CHUNK, CHUNK)], cbuf)
            pltpu.sync_copy(ew_hbm.at[_ds(base, CHUNK, CHUNK)], wbuf)
            nsub = CHUNK // GCH
            _fire(jnp.int32(0), 0)

            def sub(si, _s):
                cur = si & 1
                s0 = si * GCH

                @pl.when(si < nsub - 1)
                def _():
                    _fire(1 - cur, s0 + GCH)

                _wait(cur)

                def flushv(rowv, regs):
                    for j in range(ROW_W // 16):
                        plsc.addupdate_scatter(
                            acc_v, [rowv, lane + j * 16], regs[j])

                regs = [jnp.zeros((16,), jnp.float32)
                        for _ in range(ROW_W // 16)]
                prev_lcv = None
                bits = jnp.int32(0)
                for k in range(GCH):
                    kq, kl = divmod(k, 16)
                    if kl == 0:
                        posg = lane + jnp.broadcast_to(s0 + kq * 16, (16,))
                        colsg = plsc.load_gather(cbuf, [posg])
                        colsp = plsc.load_gather(
                            cbuf, [jnp.maximum(posg - 1, 0)])
                        bits = jnp.sum(
                            jnp.where(colsg != colsp, powv, 0))
                    posv = jnp.broadcast_to(s0 + k, (16,))
                    lcv = plsc.load_gather(cbuf, [posv]) - col0
                    wkv = plsc.load_gather(wbuf, [posv])
                    if k == 0:
                        prev_lcv = lcv
                        for j in range(ROW_W // 16):
                            regs[j] = gat2[cur, k, pl.ds(j * 16, 16)] * wkv
                        continue
                    flag = lax.shift_right_logical(bits, kl) & 1

                    @pl.when(flag != 0)
                    def _(pl_row=prev_lcv, pl_regs=list(regs)):
                        flushv(pl_row, pl_regs)

                    keep = jnp.broadcast_to(flag == 0, (16,))
                    for j in range(ROW_W // 16):
                        v = gat2[cur, k, pl.ds(j * 16, 16)] * wkv
                        regs[j] = jnp.where(keep, regs[j], 0.0) + v
                    prev_lcv = lcv
                flushv(prev_lcv, regs)
                return 0

            lax.fori_loop(0, nsub, sub, 0)

        lax.fori_loop(0, E_PAD // CHUNK, chunk_body, 0)

        r0 = pN + col0

        @pl.when(wid < NT - 1)
        def _():
            pltpu.sync_copy(acc_v, agg_hbm.at[_ds(r0, CPT, 16), :])

        @pl.when(wid == NT - 1)
        def _():
            nlast = N - (NT - 1) * CPT
            pltpu.sync_copy(acc_v.at[pl.ds(0, nlast), :],
                            agg_hbm.at[_ds(r0, nlast, 16), :])

        return 0

    lax.fori_loop(0, P, per_period, 0)


NB = 400


def _xw_body(x_ref, w2_ref, deg_ref, out_ref):
    deg = deg_ref[...]
    dis = jnp.where(deg > 0, lax.rsqrt(deg), 0.0).reshape(NB, 1)
    w2 = w2_ref[...]
    for b in range(B):
        y = jnp.dot(x_ref[0, b, :, :], w2,
                    preferred_element_type=jnp.float32,
                    precision=lax.Precision.HIGHEST)
        out_ref[0, :, b * 2 * HID:(b + 1) * 2 * HID] = y * dis


def _xw_call(xt, w2, deg):
    return pl.pallas_call(
        _xw_body,
        grid=(P, N // NB),
        in_specs=[
            pl.BlockSpec((1, B, NB, F_IN), lambda p, n: (p, 0, n, 0)),
            pl.BlockSpec((F_IN, 2 * HID), lambda p, n: (0, 0)),
            pl.BlockSpec((NB, 1), lambda p, n: (n, 0)),
        ],
        out_specs=pl.BlockSpec((1, NB, ROW_W), lambda p, n: (p, n, 0)),
        out_shape=jax.ShapeDtypeStruct((P, N, ROW_W), jnp.float32),
    )(xt, w2, deg)


ND = 400


def _out_body(agg_ref, xw_ref, deg_ref, cz_ref, ch_ref, att_ref, wl_ref,
              bl_ref, out_ref):
    att = att_ref[...]
    e = jnp.exp(att - jnp.max(att))
    probs = e / jnp.sum(e)
    deg = deg_ref[...]
    dis = jnp.where(deg > 0, lax.rsqrt(deg), 0.0).reshape(ND, 1)
    cz = cz_ref[...]
    ch = ch_ref[...]
    hacc = [jnp.zeros((ND, HID), jnp.float32) for _ in range(B)]
    for p in range(P):
        s = dis * (agg_ref[p, :, :] + dis * xw_ref[p, :, :])
        pr = probs[0:1, p:p + 1]
        for b in range(B):
            sz = s[:, b * 2 * HID:b * 2 * HID + HID]
            st = s[:, b * 2 * HID + HID:(b + 1) * 2 * HID]
            z = jax.nn.sigmoid(sz + cz)
            t = jnp.tanh(st + ch)
            hacc[b] = hacc[b] + pr * (1.0 - z) * t
    wl = wl_ref[...]
    bl = bl_ref[...]
    for b in range(B):
        h = jnp.dot(jnp.maximum(hacc[b], 0.0), wl,
                    preferred_element_type=jnp.float32,
                    precision=lax.Precision.HIGHEST)
        out_ref[b, :, :] = h + bl


def _out_call(agg, xw, deg, cz, ch, att, wlin, blin):
    return pl.pallas_call(
        _out_body,
        grid=(N // ND,),
        in_specs=[
            pl.BlockSpec((P, ND, ROW_W), lambda n: (0, n, 0)),
            pl.BlockSpec((P, ND, ROW_W), lambda n: (0, n, 0)),
            pl.BlockSpec((ND, 1), lambda n: (n, 0)),
            pl.BlockSpec((1, HID), lambda n: (0, 0)),
            pl.BlockSpec((1, HID), lambda n: (0, 0)),
            pl.BlockSpec((1, P), lambda n: (0, 0)),
            pl.BlockSpec((HID, OUT_F * P), lambda n: (0, 0)),
            pl.BlockSpec((1, OUT_F * P), lambda n: (0, 0)),
        ],
        out_specs=pl.BlockSpec((B, ND, OUT_F * P), lambda n: (0, n, 0)),
        out_shape=jax.ShapeDtypeStruct((B, N, OUT_F * P), jnp.float32),
    )(agg, xw, deg, cz, ch, att, wlin, blin)


def kernel(x, edge_index, edge_attr, attention, Wz, bz, Lz, blz, Wr, br, Lr,
           blr, Wh, bh, Lh, blh, Wlin, blin):
    row = edge_index[0]
    col = edge_index[1]
    order = jnp.argsort(col)
    rows_s = row[order]
    cols_s = col[order]
    ew_s = edge_attr[order]
    tb0 = jnp.searchsorted(
        cols_s, jnp.arange(NT + 1, dtype=jnp.int32) * CPT).astype(jnp.int32)
    counts = tb0[1:] - tb0[:-1]
    pc = ((counts + CHUNK - 1) // CHUNK) * CHUNK
    off = jnp.concatenate(
        [jnp.zeros((1,), jnp.int32), jnp.cumsum(pc).astype(jnp.int32)])
    slot = jnp.arange(E_PAD, dtype=jnp.int32)
    slot_tile = jnp.clip(
        jnp.searchsorted(off[1:], slot, side="right"), 0, NT - 1)
    cols_p = (slot_tile * CPT).astype(jnp.int32)
    rows_p = jnp.zeros((E_PAD,), jnp.int32)
    ew_p = jnp.zeros((E_PAD,), jnp.float32)
    tile_id = cols_s // CPT
    newpos = off[tile_id] + jnp.arange(E, dtype=jnp.int32) - tb0[tile_id]
    cols_p = cols_p.at[newpos].set(cols_s)
    rows_p = rows_p.at[newpos].set(rows_s)
    ew_p = ew_p.at[newpos].set(ew_s)
    tb = jnp.concatenate(
        [off, jnp.full((40 - NT - 1,), off[NT], jnp.int32)])

    w2 = jnp.concatenate([Wz @ Lz[:HID], Wh @ Lh[:HID]], axis=1)
    cz = (bz @ Lz[:HID] + blz).reshape(1, HID)
    ch = (bh @ Lh[:HID] + blh).reshape(1, HID)

    deg = _deg_kernel(cols_p, ew_p, tb).reshape(N, 1)

    xt = jnp.transpose(x, (3, 0, 1, 2))
    xw = _xw_call(xt, w2, deg)

    agg = _agg_kernel(xw.reshape(P * N, ROW_W), rows_p, cols_p, ew_p, tb)

    out = _out_call(agg.reshape(P, N, ROW_W), xw, deg, cz, ch,
                    attention.reshape(1, P), Wlin, blin.reshape(1, OUT_F * P))
    return out.reshape(B, N, OUT_F, P)


# speedup vs baseline: 1.4044x; 1.4044x over previous
import functools

import jax
import jax.numpy as jnp
from jax import lax
from jax.experimental import pallas as pl
from jax.experimental.pallas import tpu as pltpu
from jax.experimental.pallas import tpu_sc as plsc

B = 4
N = 10000
E = 320000
F_IN = 128
HID = 32
P = 12
OUT_F = 2

NT = 32
CPT = 320
CHUNK = 2048
GCH = 64
ROW_W = B * 2 * HID
E_PAD = ((E + CHUNK - 1) // CHUNK) * CHUNK

_mesh = plsc.VectorSubcoreMesh(core_axis_name="c", subcore_axis_name="s")
_sc_params = pltpu.CompilerParams(needs_layout_passes=False)


def _wid():
    return lax.axis_index("s") * 2 + lax.axis_index("c")


def _ds(start, size, mult=8):
    return pl.ds(pl.multiple_of(start, mult), size)


@functools.partial(
    pl.kernel,
    out_type=jax.ShapeDtypeStruct((N,), jnp.float32),
    mesh=_mesh,
    compiler_params=_sc_params,
    scratch_types=[
        pltpu.VMEM((40,), jnp.int32),
        pltpu.VMEM((CPT,), jnp.float32),
        pltpu.VMEM((CHUNK,), jnp.int32),
        pltpu.VMEM((CHUNK,), jnp.float32),
    ],
)
def _deg_kernel(cols_hbm, ew_hbm, tb_hbm, deg_hbm, tb_v, deg_v, cbuf,
                wbuf):
    wid = _wid()
    col0 = wid * CPT
    pltpu.sync_copy(tb_hbm, tb_v)
    lane = lax.iota(jnp.int32, 16)
    lane0 = lane == 0
    widv = jnp.broadcast_to(wid, (16,))
    tsv = plsc.load_gather(tb_v, [widv])
    tev = plsc.load_gather(tb_v, [widv + 1])
    ts = jnp.max(tsv)
    te = jnp.max(tev)

    for i in range(CPT // 16):
        deg_v[pl.ds(i * 16, 16)] = jnp.ones((16,), jnp.float32)

    def chunk_body(ci, _c):
        base = ci * CHUNK

        @pl.when((base + CHUNK > ts) & (base < te))
        def _():
            pltpu.sync_copy(cols_hbm.at[_ds(base, CHUNK, CHUNK)], cbuf)
            pltpu.sync_copy(ew_hbm.at[_ds(base, CHUNK, CHUNK)], wbuf)

            def grp(gi, _):
                g0 = gi * 16
                for k in range(16):
                    posv = jnp.broadcast_to(g0 + k, (16,))
                    ck = plsc.load_gather(cbuf, [posv]) - col0
                    wk = plsc.load_gather(wbuf, [posv])
                    gv = posv + base
                    mk = (gv >= tsv) & (gv < tev) & lane0
                    plsc.addupdate_scatter(deg_v, [ck], wk, mask=mk)
                return 0

            lax.fori_loop(0, CHUNK // 16, grp, 0)

        return 0

    lax.fori_loop(0, E_PAD // CHUNK, chunk_body, 0)

    @pl.when(wid < NT - 1)
    def _():
        pltpu.sync_copy(deg_v, deg_hbm.at[_ds(col0, CPT, CPT)])

    @pl.when(wid == NT - 1)
    def _():
        pltpu.sync_copy(deg_v.at[pl.ds(0, N - (NT - 1) * CPT)],
                        deg_hbm.at[_ds(col0, N - (NT - 1) * CPT, 16)])


@functools.partial(
    pl.kernel,
    out_type=jax.ShapeDtypeStruct((P * N, ROW_W), jnp.float32),
    mesh=_mesh,
    compiler_params=_sc_params,
    scratch_types=[
        pltpu.VMEM((40,), jnp.int32),
        pltpu.VMEM((CPT, ROW_W), jnp.float32),
        pltpu.VMEM((CHUNK,), jnp.int32),
        pltpu.VMEM((CHUNK,), jnp.int32),
        pltpu.VMEM((CHUNK,), jnp.float32),
        pltpu.VMEM((2, GCH), jnp.int32),
        pltpu.VMEM((2, GCH, ROW_W), jnp.float32),
        pltpu.SemaphoreType.DMA,
        pltpu.SemaphoreType.DMA,
    ],
)
def _agg_kernel(xw_hbm, rows_hbm, cols_hbm, ew_hbm, tb_hbm, agg_hbm,
                tb_v, acc_v, rbuf, cbuf, wbuf, idx2, gat2, sem0, sem1):
    wid = _wid()
    col0 = wid * CPT
    pltpu.sync_copy(tb_hbm, tb_v)
    lane = lax.iota(jnp.int32, 16)
    powv = jnp.int32(1) << lane
    zeros16 = jnp.zeros((16,), jnp.float32)
    widv = jnp.broadcast_to(wid, (16,))
    tsv = plsc.load_gather(tb_v, [widv])
    tev = plsc.load_gather(tb_v, [widv + 1])
    ts = jnp.max(tsv)
    te = jnp.max(tev)

    def per_period(p, _):
        pN = p * N

        def zrow(i, _z):
            for j in range(ROW_W // 16):
                acc_v[i, pl.ds(j * 16, 16)] = zeros16
            return 0

        lax.fori_loop(0, CPT, zrow, 0)

        def chunk_body(ci, _c):
            base = ci * CHUNK

            @pl.when((base + CHUNK > ts) & (base < te))
            def _():
                _chunk_inner(base)
            return 0

        def _fire(buf, s0):
            for q in range(GCH // 16):
                idx2[buf, pl.ds(q * 16, 16)] = (
                    rbuf[_ds(s0 + q * 16, 16, 16)] + pN)

            @pl.when(buf == 0)
            def _():
                pltpu.async_copy(xw_hbm.at[idx2.at[0]], gat2.at[0], sem0)

            @pl.when(buf == 1)
            def _():
                pltpu.async_copy(xw_hbm.at[idx2.at[1]], gat2.at[1], sem1)

        def _wait(buf):
            @pl.when(buf == 0)
            def _():
                pltpu.make_async_copy(
                    xw_hbm.at[idx2.at[0]], gat2.at[0], sem0).wait()

            @pl.when(buf == 1)
            def _():
                pltpu.make_async_copy(
                    xw_hbm.at[idx2.at[1]], gat2.at[1], sem1).wait()

        def _chunk_inner(base):
            pltpu.sync_copy(rows_hbm.at[_ds(base, CHUNK, CHUNK)], rbuf)
            pltpu.sync_copy(cols_hbm.at[_ds(base, CHUNK, CHUNK)], cbuf)
            pltpu.sync_copy(ew_hbm.at[_ds(base, CHUNK, CHUNK)], wbuf)
            nsub = CHUNK // GCH
            _fire(jnp.int32(0), 0)

            def sub(si, _s):
                cur = si & 1
                s0 = si * GCH

                @pl.when(si < nsub - 1)
                def _():
                    _fire(1 - cur, s0 + GCH)

                _wait(cur)

                def flushm(rowv, regs, m):
                    for j in range(ROW_W // 16):
                        plsc.addupdate_scatter(
                            acc_v, [rowv, lane + j * 16], regs[j], mask=m)

                regs = [jnp.zeros((16,), jnp.float32)
                        for _ in range(ROW_W // 16)]
                prev_lcv = None
                for k in range(GCH):
                    posv = jnp.broadcast_to(s0 + k, (16,))
                    ckv = plsc.load_gather(cbuf, [posv]) - col0
                    wkv = plsc.load_gather(wbuf, [posv])
                    gv = posv + base
                    mk = (gv >= tsv) & (gv < tev)
                    wkv = jnp.where(mk, wkv, 0.0)
                    lcv = jnp.clip(ckv, 0, CPT - 1)
                    if k == 0:
                        prev_lcv = lcv
                        for j in range(ROW_W // 16):
                            regs[j] = gat2[cur, k, pl.ds(j * 16, 16)] * wkv
                        continue
                    neq = lcv != prev_lcv
                    flushm(prev_lcv, regs, neq)
                    keepf = jnp.where(neq, 0.0, 1.0)
                    for j in range(ROW_W // 16):
                        v = gat2[cur, k, pl.ds(j * 16, 16)] * wkv
                        regs[j] = regs[j] * keepf + v
                    prev_lcv = lcv
                flushm(prev_lcv, regs, None)
                return 0

            lax.fori_loop(0, nsub, sub, 0)

        lax.fori_loop(0, E_PAD // CHUNK, chunk_body, 0)

        r0 = pN + col0

        @pl.when(wid < NT - 1)
        def _():
            pltpu.sync_copy(acc_v, agg_hbm.at[_ds(r0, CPT, 16), :])

        @pl.when(wid == NT - 1)
        def _():
            nlast = N - (NT - 1) * CPT
            pltpu.sync_copy(acc_v.at[pl.ds(0, nlast), :],
                            agg_hbm.at[_ds(r0, nlast, 16), :])

        return 0

    lax.fori_loop(0, P, per_period, 0)


NB = 400


def _xw_body(x_ref, w2_ref, deg_ref, out_ref):
    deg = deg_ref[...]
    dis = jnp.where(deg > 0, lax.rsqrt(deg), 0.0).reshape(NB, 1)
    w2 = w2_ref[...]
    for b in range(B):
        y = jnp.dot(x_ref[0, b, :, :], w2,
                    preferred_element_type=jnp.float32,
                    precision=lax.Precision.HIGHEST)
        out_ref[0, :, b * 2 * HID:(b + 1) * 2 * HID] = y * dis


def _xw_call(xt, w2, deg):
    return pl.pallas_call(
        _xw_body,
        grid=(P, N // NB),
        in_specs=[
            pl.BlockSpec((1, B, NB, F_IN), lambda p, n: (p, 0, n, 0)),
            pl.BlockSpec((F_IN, 2 * HID), lambda p, n: (0, 0)),
            pl.BlockSpec((NB, 1), lambda p, n: (n, 0)),
        ],
        out_specs=pl.BlockSpec((1, NB, ROW_W), lambda p, n: (p, n, 0)),
        out_shape=jax.ShapeDtypeStruct((P, N, ROW_W), jnp.float32),
    )(xt, w2, deg)


ND = 400


def _out_body(agg_ref, xw_ref, deg_ref, cz_ref, ch_ref, att_ref, wl_ref,
              bl_ref, out_ref):
    att = att_ref[...]
    e = jnp.exp(att - jnp.max(att))
    probs = e / jnp.sum(e)
    deg = deg_ref[...]
    dis = jnp.where(deg > 0, lax.rsqrt(deg), 0.0).reshape(ND, 1)
    cz = cz_ref[...]
    ch = ch_ref[...]
    hacc = [jnp.zeros((ND, HID), jnp.float32) for _ in range(B)]
    for p in range(P):
        s = dis * (agg_ref[p, :, :] + dis * xw_ref[p, :, :])
        pr = probs[0:1, p:p + 1]
        for b in range(B):
            sz = s[:, b * 2 * HID:b * 2 * HID + HID]
            st = s[:, b * 2 * HID + HID:(b + 1) * 2 * HID]
            z = jax.nn.sigmoid(sz + cz)
            t = jnp.tanh(st + ch)
            hacc[b] = hacc[b] + pr * (1.0 - z) * t
    wl = wl_ref[...]
    bl = bl_ref[...]
    for b in range(B):
        h = jnp.dot(jnp.maximum(hacc[b], 0.0), wl,
                    preferred_element_type=jnp.float32,
                    precision=lax.Precision.HIGHEST)
        out_ref[b, :, :] = h + bl


def _out_call(agg, xw, deg, cz, ch, att, wlin, blin):
    return pl.pallas_call(
        _out_body,
        grid=(N // ND,),
        in_specs=[
            pl.BlockSpec((P, ND, ROW_W), lambda n: (0, n, 0)),
            pl.BlockSpec((P, ND, ROW_W), lambda n: (0, n, 0)),
            pl.BlockSpec((ND, 1), lambda n: (n, 0)),
            pl.BlockSpec((1, HID), lambda n: (0, 0)),
            pl.BlockSpec((1, HID), lambda n: (0, 0)),
            pl.BlockSpec((1, P), lambda n: (0, 0)),
            pl.BlockSpec((HID, OUT_F * P), lambda n: (0, 0)),
            pl.BlockSpec((1, OUT_F * P), lambda n: (0, 0)),
        ],
        out_specs=pl.BlockSpec((B, ND, OUT_F * P), lambda n: (0, n, 0)),
        out_shape=jax.ShapeDtypeStruct((B, N, OUT_F * P), jnp.float32),
    )(agg, xw, deg, cz, ch, att, wlin, blin)


def kernel(x, edge_index, edge_attr, attention, Wz, bz, Lz, blz, Wr, br, Lr,
           blr, Wh, bh, Lh, blh, Wlin, blin):
    row = edge_index[0]
    col = edge_index[1]
    order = jnp.argsort(col)
    rows_p = row[order]
    cols_p = col[order]
    ew_p = edge_attr[order]
    tb = jnp.searchsorted(
        cols_p, jnp.arange(NT + 1, dtype=jnp.int32) * CPT).astype(jnp.int32)
    tb = jnp.concatenate([tb, jnp.full((40 - NT - 1,), E, jnp.int32)])
    rows_p = jnp.concatenate([rows_p, jnp.zeros((E_PAD - E,), jnp.int32)])
    cols_p = jnp.concatenate([cols_p, jnp.zeros((E_PAD - E,), jnp.int32)])
    ew_p = jnp.concatenate([ew_p, jnp.zeros((E_PAD - E,), jnp.float32)])

    w2 = jnp.concatenate([Wz @ Lz[:HID], Wh @ Lh[:HID]], axis=1)
    cz = (bz @ Lz[:HID] + blz).reshape(1, HID)
    ch = (bh @ Lh[:HID] + blh).reshape(1, HID)

    deg = _deg_kernel(cols_p, ew_p, tb).reshape(N, 1)

    xt = jnp.transpose(x, (3, 0, 1, 2))
    xw = _xw_call(xt, w2, deg)

    agg = _agg_kernel(xw.reshape(P * N, ROW_W), rows_p, cols_p, ew_p, tb)

    out = _out_call(agg.reshape(P, N, ROW_W), xw, deg, cz, ch,
                    attention.reshape(1, P), Wlin, blin.reshape(1, OUT_F * P))
    return out.reshape(B, N, OUT_F, P)


# speedup vs baseline: 1.5577x; 1.1092x over previous
import functools

import jax
import jax.numpy as jnp
from jax import lax
from jax.experimental import pallas as pl
from jax.experimental.pallas import tpu as pltpu
from jax.experimental.pallas import tpu_sc as plsc

B = 4
N = 10000
E = 320000
F_IN = 128
HID = 32
P = 12
OUT_F = 2

NT = 32
CPT = 320
CHUNK = 2048
GCH = 64
ROW_W = B * 2 * HID
E_PAD = ((E + CHUNK - 1) // CHUNK) * CHUNK

_mesh = plsc.VectorSubcoreMesh(core_axis_name="c", subcore_axis_name="s")
_sc_params = pltpu.CompilerParams(needs_layout_passes=False)


def _wid():
    return lax.axis_index("s") * 2 + lax.axis_index("c")


def _ds(start, size, mult=8):
    return pl.ds(pl.multiple_of(start, mult), size)


@functools.partial(
    pl.kernel,
    out_type=jax.ShapeDtypeStruct((N,), jnp.float32),
    mesh=_mesh,
    compiler_params=_sc_params,
    scratch_types=[
        pltpu.VMEM((40,), jnp.int32),
        pltpu.VMEM((CPT,), jnp.float32),
        pltpu.VMEM((CHUNK,), jnp.int32),
        pltpu.VMEM((CHUNK,), jnp.float32),
    ],
)
def _deg_kernel(cols_hbm, ew_hbm, tb_hbm, deg_hbm, tb_v, deg_v, cbuf,
                wbuf):
    wid = _wid()
    col0 = wid * CPT
    pltpu.sync_copy(tb_hbm, tb_v)
    lane = lax.iota(jnp.int32, 16)
    lane0 = lane == 0
    widv = jnp.broadcast_to(wid, (16,))
    tsv = plsc.load_gather(tb_v, [widv])
    tev = plsc.load_gather(tb_v, [widv + 1])
    ts = jnp.max(tsv)
    te = jnp.max(tev)

    for i in range(CPT // 16):
        deg_v[pl.ds(i * 16, 16)] = jnp.ones((16,), jnp.float32)

    def chunk_body(ci, _c):
        base = ci * CHUNK

        @pl.when((base + CHUNK > ts) & (base < te))
        def _():
            pltpu.sync_copy(cols_hbm.at[_ds(base, CHUNK, CHUNK)], cbuf)
            pltpu.sync_copy(ew_hbm.at[_ds(base, CHUNK, CHUNK)], wbuf)

            def grp(gi, _):
                g0 = gi * 16
                for k in range(16):
                    posv = jnp.broadcast_to(g0 + k, (16,))
                    ck = plsc.load_gather(cbuf, [posv]) - col0
                    wk = plsc.load_gather(wbuf, [posv])
                    gv = posv + base
                    mk = (gv >= tsv) & (gv < tev) & lane0
                    plsc.addupdate_scatter(deg_v, [ck], wk, mask=mk)
                return 0

            lax.fori_loop(0, CHUNK // 16, grp, 0)

        return 0

    lax.fori_loop(0, E_PAD // CHUNK, chunk_body, 0)

    @pl.when(wid < NT - 1)
    def _():
        pltpu.sync_copy(deg_v, deg_hbm.at[_ds(col0, CPT, CPT)])

    @pl.when(wid == NT - 1)
    def _():
        pltpu.sync_copy(deg_v.at[pl.ds(0, N - (NT - 1) * CPT)],
                        deg_hbm.at[_ds(col0, N - (NT - 1) * CPT, 16)])


@functools.partial(
    pl.kernel,
    out_type=jax.ShapeDtypeStruct((P * N * ROW_W,), jnp.float32),
    mesh=_mesh,
    compiler_params=_sc_params,
    scratch_types=[
        pltpu.VMEM((40,), jnp.int32),
        pltpu.VMEM((CPT * ROW_W,), jnp.float32),
        pltpu.VMEM((CHUNK,), jnp.int32),
        pltpu.VMEM((CHUNK,), jnp.int32),
        pltpu.VMEM((CHUNK,), jnp.float32),
        pltpu.VMEM((2, GCH), jnp.int32),
        pltpu.VMEM((2, GCH, ROW_W), jnp.float32),
        pltpu.SemaphoreType.DMA,
        pltpu.SemaphoreType.DMA,
    ],
)
def _agg_kernel(xw_hbm, rows_hbm, cols_hbm, ew_hbm, tb_hbm, agg_hbm,
                tb_v, acc_v, rbuf, cbuf, wbuf, idx2, gat2, sem0, sem1):
    wid = _wid()
    col0 = wid * CPT
    pltpu.sync_copy(tb_hbm, tb_v)
    lane = lax.iota(jnp.int32, 16)
    powv = jnp.int32(1) << lane
    zeros16 = jnp.zeros((16,), jnp.float32)
    widv = jnp.broadcast_to(wid, (16,))
    tsv = plsc.load_gather(tb_v, [widv])
    tev = plsc.load_gather(tb_v, [widv + 1])
    ts = jnp.max(tsv)
    te = jnp.max(tev)

    def per_period(p, _):
        pN = p * N

        def zrow(i, _z):
            for j in range(ROW_W // 16):
                acc_v[_ds(i * ROW_W + j * 16, 16, 16)] = zeros16
            return 0

        lax.fori_loop(0, CPT, zrow, 0)

        def chunk_body(ci, _c):
            base = ci * CHUNK

            @pl.when((base + CHUNK > ts) & (base < te))
            def _():
                _chunk_inner(base)
            return 0

        def _fire(buf, s0):
            for q in range(GCH // 16):
                idx2[buf, pl.ds(q * 16, 16)] = (
                    rbuf[_ds(s0 + q * 16, 16, 16)] + pN)

            @pl.when(buf == 0)
            def _():
                pltpu.async_copy(xw_hbm.at[idx2.at[0]], gat2.at[0], sem0)

            @pl.when(buf == 1)
            def _():
                pltpu.async_copy(xw_hbm.at[idx2.at[1]], gat2.at[1], sem1)

        def _wait(buf):
            @pl.when(buf == 0)
            def _():
                pltpu.make_async_copy(
                    xw_hbm.at[idx2.at[0]], gat2.at[0], sem0).wait()

            @pl.when(buf == 1)
            def _():
                pltpu.make_async_copy(
                    xw_hbm.at[idx2.at[1]], gat2.at[1], sem1).wait()

        def _chunk_inner(base):
            pltpu.sync_copy(rows_hbm.at[_ds(base, CHUNK, CHUNK)], rbuf)
            pltpu.sync_copy(cols_hbm.at[_ds(base, CHUNK, CHUNK)], cbuf)
            pltpu.sync_copy(ew_hbm.at[_ds(base, CHUNK, CHUNK)], wbuf)
            nsub = CHUNK // GCH
            _fire(jnp.int32(0), 0)

            def sub(si, _s):
                cur = si & 1
                s0 = si * GCH

                @pl.when(si < nsub - 1)
                def _():
                    _fire(1 - cur, s0 + GCH)

                _wait(cur)

                def flushm(basev, regs, m):
                    for j in range(ROW_W // 16):
                        plsc.addupdate_scatter(
                            acc_v, [basev + j * 16], regs[j], mask=m)

                gref = gat2.at[cur]
                regs = [jnp.zeros((16,), jnp.float32)
                        for _ in range(ROW_W // 16)]
                prev_lcv = None
                prev_base = None
                for k in range(GCH):
                    posv = jnp.broadcast_to(s0 + k, (16,))
                    ckv = plsc.load_gather(cbuf, [posv]) - col0
                    wkv = plsc.load_gather(wbuf, [posv])
                    gv = posv + base
                    mk = (gv >= tsv) & (gv < tev)
                    wkv = jnp.where(mk, wkv, 0.0)
                    lcv = jnp.clip(ckv, 0, CPT - 1)
                    if k == 0:
                        prev_lcv = lcv
                        prev_base = lcv * ROW_W + lane
                        for j in range(ROW_W // 16):
                            regs[j] = gref[k, pl.ds(j * 16, 16)] * wkv
                        continue
                    neq = lcv != prev_lcv
                    flushm(prev_base, regs, neq)
                    keepf = jnp.where(neq, 0.0, 1.0)
                    for j in range(ROW_W // 16):
                        v = gref[k, pl.ds(j * 16, 16)] * wkv
                        regs[j] = regs[j] * keepf + v
                    prev_lcv = lcv
                    prev_base = lcv * ROW_W + lane
                flushm(prev_base, regs, None)
                return 0

            lax.fori_loop(0, nsub, sub, 0)

        lax.fori_loop(0, E_PAD // CHUNK, chunk_body, 0)

        r0 = (pN + col0) * ROW_W

        @pl.when(wid < NT - 1)
        def _():
            pltpu.sync_copy(acc_v, agg_hbm.at[_ds(r0, CPT * ROW_W, 256)])

        @pl.when(wid == NT - 1)
        def _():
            nlast = N - (NT - 1) * CPT
            pltpu.sync_copy(acc_v.at[pl.ds(0, nlast * ROW_W)],
                            agg_hbm.at[_ds(r0, nlast * ROW_W, 256)])

        return 0

    lax.fori_loop(0, P, per_period, 0)


NB = 400


def _xw_body(x_ref, w2_ref, deg_ref, out_ref):
    deg = deg_ref[...]
    dis = jnp.where(deg > 0, lax.rsqrt(deg), 0.0).reshape(NB, 1)
    w2 = w2_ref[...]
    for b in range(B):
        y = jnp.dot(x_ref[0, b, :, :], w2,
                    preferred_element_type=jnp.float32,
                    precision=lax.Precision.HIGHEST)
        out_ref[0, :, b * 2 * HID:(b + 1) * 2 * HID] = y * dis


def _xw_call(xt, w2, deg):
    return pl.pallas_call(
        _xw_body,
        grid=(P, N // NB),
        in_specs=[
            pl.BlockSpec((1, B, NB, F_IN), lambda p, n: (p, 0, n, 0)),
            pl.BlockSpec((F_IN, 2 * HID), lambda p, n: (0, 0)),
            pl.BlockSpec((NB, 1), lambda p, n: (n, 0)),
        ],
        out_specs=pl.BlockSpec((1, NB, ROW_W), lambda p, n: (p, n, 0)),
        out_shape=jax.ShapeDtypeStruct((P, N, ROW_W), jnp.float32),
    )(xt, w2, deg)


ND = 400


def _out_body(agg_ref, xw_ref, deg_ref, cz_ref, ch_ref, att_ref, wl_ref,
              bl_ref, out_ref):
    att = att_ref[...]
    e = jnp.exp(att - jnp.max(att))
    probs = e / jnp.sum(e)
    deg = deg_ref[...]
    dis = jnp.where(deg > 0, lax.rsqrt(deg), 0.0).reshape(ND, 1)
    cz = cz_ref[...]
    ch = ch_ref[...]
    hacc = [jnp.zeros((ND, HID), jnp.float32) for _ in range(B)]
    for p in range(P):
        s = dis * (agg_ref[p, :, :] + dis * xw_ref[p, :, :])
        pr = probs[0:1, p:p + 1]
        for b in range(B):
            sz = s[:, b * 2 * HID:b * 2 * HID + HID]
            st = s[:, b * 2 * HID + HID:(b + 1) * 2 * HID]
            z = jax.nn.sigmoid(sz + cz)
            t = jnp.tanh(st + ch)
            hacc[b] = hacc[b] + pr * (1.0 - z) * t
    wl = wl_ref[...]
    bl = bl_ref[...]
    for b in range(B):
        h = jnp.dot(jnp.maximum(hacc[b], 0.0), wl,
                    preferred_element_type=jnp.float32,
                    precision=lax.Precision.HIGHEST)
        out_ref[b, :, :] = h + bl


def _out_call(agg, xw, deg, cz, ch, att, wlin, blin):
    return pl.pallas_call(
        _out_body,
        grid=(N // ND,),
        in_specs=[
            pl.BlockSpec((P, ND, ROW_W), lambda n: (0, n, 0)),
            pl.BlockSpec((P, ND, ROW_W), lambda n: (0, n, 0)),
            pl.BlockSpec((ND, 1), lambda n: (n, 0)),
            pl.BlockSpec((1, HID), lambda n: (0, 0)),
            pl.BlockSpec((1, HID), lambda n: (0, 0)),
            pl.BlockSpec((1, P), lambda n: (0, 0)),
            pl.BlockSpec((HID, OUT_F * P), lambda n: (0, 0)),
            pl.BlockSpec((1, OUT_F * P), lambda n: (0, 0)),
        ],
        out_specs=pl.BlockSpec((B, ND, OUT_F * P), lambda n: (0, n, 0)),
        out_shape=jax.ShapeDtypeStruct((B, N, OUT_F * P), jnp.float32),
    )(agg, xw, deg, cz, ch, att, wlin, blin)


def kernel(x, edge_index, edge_attr, attention, Wz, bz, Lz, blz, Wr, br, Lr,
           blr, Wh, bh, Lh, blh, Wlin, blin):
    row = edge_index[0]
    col = edge_index[1]
    order = jnp.argsort(col)
    rows_p = row[order]
    cols_p = col[order]
    ew_p = edge_attr[order]
    tb = jnp.searchsorted(
        cols_p, jnp.arange(NT + 1, dtype=jnp.int32) * CPT).astype(jnp.int32)
    tb = jnp.concatenate([tb, jnp.full((40 - NT - 1,), E, jnp.int32)])
    rows_p = jnp.concatenate([rows_p, jnp.zeros((E_PAD - E,), jnp.int32)])
    cols_p = jnp.concatenate([cols_p, jnp.zeros((E_PAD - E,), jnp.int32)])
    ew_p = jnp.concatenate([ew_p, jnp.zeros((E_PAD - E,), jnp.float32)])

    w2 = jnp.concatenate([Wz @ Lz[:HID], Wh @ Lh[:HID]], axis=1)
    cz = (bz @ Lz[:HID] + blz).reshape(1, HID)
    ch = (bh @ Lh[:HID] + blh).reshape(1, HID)

    deg = _deg_kernel(cols_p, ew_p, tb).reshape(N, 1)

    xt = jnp.transpose(x, (3, 0, 1, 2))
    xw = _xw_call(xt, w2, deg)

    agg = _agg_kernel(xw.reshape(P * N, ROW_W), rows_p, cols_p, ew_p, tb)

    out = _out_call(agg.reshape(P, N, ROW_W), xw, deg, cz, ch,
                    attention.reshape(1, P), Wlin, blin.reshape(1, OUT_F * P))
    return out.reshape(B, N, OUT_F, P)
